# baseline (speedup 1.0000x reference)
import functools

import jax
import jax.numpy as jnp
from jax import lax
from jax.experimental import pallas as pl
from jax.experimental.pallas import tpu as pltpu
from jax.experimental.pallas import tpu_sc as plsc

_SC_CH = 32768


def _sc_busy_kernel():
    mesh = plsc.VectorSubcoreMesh(core_axis_name="c", subcore_axis_name="s")

    @functools.partial(
        pl.kernel, mesh=mesh,
        out_type=jax.ShapeDtypeStruct((32, 16), jnp.float32),
        scratch_types=[pltpu.VMEM((_SC_CH,), jnp.float32),
                       pltpu.VMEM((16,), jnp.float32)])
    def sc_busy(x_hbm, out_hbm, buf, out_v):
        wid = lax.axis_index("s") * 2 + lax.axis_index("c")
        pltpu.sync_copy(x_hbm.at[pl.ds(wid * _SC_CH, _SC_CH)], buf)

        def body(i, acc):
            return acc + buf[pl.ds((i % (_SC_CH // 16)) * 16, 16)]

        acc = jax.lax.fori_loop(0, 8 * (_SC_CH // 16), body,
                                jnp.zeros((16,), jnp.float32))
        out_v[...] = acc
        pltpu.sync_copy(out_v, out_hbm.at[wid])

    return sc_busy

_ALPHA = 0.25
_POS_THR = 0.5
_NEG_THR = 0.4
_BETA = 1.0 / 9

_B, _N, _C, _M = 8, 20000, 80, 50
_K = 2048
_CHUNKS = [(i * _K, _K) for i in range(9)] + [(9 * _K, _N - 9 * _K)]


def _iou_tile(a, ann_cols):
    gx1, gy1, gx2, gy2, area_g = ann_cols
    ax1 = a[0:1, :]
    ay1 = a[1:2, :]
    ax2 = a[2:3, :]
    ay2 = a[3:4, :]
    area_a = (ax2 - ax1) * (ay2 - ay1)
    ltx = jnp.maximum(ax1, gx1)
    lty = jnp.maximum(ay1, gy1)
    rbx = jnp.minimum(ax2, gx2)
    rby = jnp.minimum(ay2, gy2)
    whx = jnp.maximum(rbx - ltx, 0.0)
    why = jnp.maximum(rby - lty, 0.0)
    inter = whx * why
    union = area_a + area_g - inter
    return inter / jnp.maximum(union, 1e-6)


def _body(cls_ref, reg_ref, anc_ref, ann_ref, annT_ref, outc_ref, outr_ref,
          ov_scr):
    b = pl.program_id(0)

    ann = ann_ref[0]
    annT = annT_ref[0]
    gx1 = ann[:, 0:1]
    gy1 = ann[:, 1:2]
    gx2 = ann[:, 2:3]
    gy2 = ann[:, 3:4]
    area_g = (gx2 - gx1) * (gy2 - gy1)
    ann_cols = (gx1, gy1, gx2, gy2, area_g)

    gcls = ann[:, 4:5].astype(jnp.int32)
    gmat = (jax.lax.broadcasted_iota(jnp.int32, (_M, _C), 1) == gcls
            ).astype(jnp.bfloat16)
    ones_c = jnp.full((1, _C), 1.0, dtype=jnp.bfloat16)

    gt_max = jnp.full((_M, 1), -1.0, dtype=jnp.float32)
    for off, k in _CHUNKS:
        ov = _iou_tile(anc_ref[:, off:off + k], ann_cols)
        ov_scr[:, off:off + k] = ov
        gt_max = jnp.maximum(gt_max, jnp.max(ov, axis=1, keepdims=True))

    cls_acc = jnp.float32(0.0)
    reg_acc = jnp.float32(0.0)
    np_acc = jnp.float32(0.0)
    for off, k in _CHUNKS:
        iota_m = jax.lax.broadcasted_iota(jnp.int32, (_M, k), 0)
        ov = ov_scr[:, off:off + k]
        maxov = jnp.max(ov, axis=0, keepdims=True)
        eq = ov == maxov
        amax = jnp.min(jnp.where(eq, iota_m, _M), axis=0, keepdims=True)
        lq = jnp.any(ov == gt_max, axis=0, keepdims=True)
        pos05 = maxov >= _POS_THR
        basef = (pos05 | (maxov < _NEG_THR)).astype(jnp.float32)
        posf = (pos05 | lq).astype(jnp.float32)

        onehot_m = (iota_m == amax).astype(jnp.float32)
        assigned = jax.lax.dot_general(
            annT, onehot_m, (((1,), (0,)), ((), ())),
            preferred_element_type=jnp.float32)

        p = jnp.clip(cls_ref[0, :, off:off + k], 1e-4, 1.0 - 1e-4
                     ).astype(jnp.bfloat16)
        l0 = (p * p) * (jnp.bfloat16(-0.75) *
                        jnp.log(jnp.bfloat16(1.0) - p))
        s_bg = jax.lax.dot_general(
            ones_c, l0, (((1,), (0,)), ((), ())),
            preferred_element_type=jnp.float32)
        q = jax.lax.dot_general(
            gmat, p, (((1,), (0,)), ((), ())),
            preferred_element_type=jnp.float32)
        sel = jnp.sum(q * onehot_m, axis=0, keepdims=True)
        l0_l = (sel * sel) * (-0.75 * jnp.log(1.000001 - sel))
        oms = 1.0 - sel
        l1_l = (oms * oms) * (-0.25 * jnp.log(sel + 1e-6))
        cls_acc += jnp.sum(basef * s_bg + posf * l1_l - (posf * basef) * l0_l)
        np_acc += jnp.sum(posf)

        a = anc_ref[:, off:off + k]
        aw = a[2:3, :] - a[0:1, :]
        ah = a[3:4, :] - a[1:2, :]
        axc = a[0:1, :] + 0.5 * aw
        ayc = a[1:2, :] + 0.5 * ah
        gw = assigned[2:3, :] - assigned[0:1, :]
        gh = assigned[3:4, :] - assigned[1:2, :]
        gxc = assigned[0:1, :] + 0.5 * gw
        gyc = assigned[1:2, :] + 0.5 * gh
        r = reg_ref[0, :, off:off + k]
        d0 = jnp.abs(r[0:1, :] - (gxc - axc) / aw)
        d1 = jnp.abs(r[1:2, :] - (gyc - ayc) / ah)
        d2 = jnp.abs(r[2:3, :] - jnp.log(gw / aw))
        d3 = jnp.abs(r[3:4, :] - jnp.log(gh / ah))

        def _sl1(d):
            return jnp.where(d < _BETA, 0.5 * d * d / _BETA, d - 0.5 * _BETA)

        reg_acc += jnp.sum(posf * (_sl1(d0) + _sl1(d1) + _sl1(d2) + _sl1(d3)))

    cls_img = cls_acc / jnp.maximum(np_acc, 1.0)
    reg_img = jnp.where(np_acc > 0.0,
                        reg_acc / jnp.maximum(np_acc * 4.0, 1.0), 0.0)
    cls_v = jnp.reshape(cls_img * 0.125, (1, 1))
    reg_v = jnp.reshape(reg_img * 0.125, (1, 1))

    @pl.when(b == 0)
    def _():
        outc_ref[:, :] = cls_v
        outr_ref[:, :] = reg_v

    @pl.when(b != 0)
    def _():
        outc_ref[:, :] += cls_v
        outr_ref[:, :] += reg_v


@functools.partial(jax.jit, static_argnames=("interpret",))
def _run(classifications, regressions, anchors, annotations, interpret=False):
    clsT = jnp.transpose(classifications, (0, 2, 1))
    regT = jnp.transpose(regressions, (0, 2, 1))
    ancT = jnp.transpose(anchors[0])
    annT = jnp.transpose(annotations, (0, 2, 1))

    outc, outr = pl.pallas_call(
        _body,
        grid=(_B,),
        in_specs=[
            pl.BlockSpec((1, _C, _N), lambda b: (b, 0, 0)),
            pl.BlockSpec((1, 4, _N), lambda b: (b, 0, 0)),
            pl.BlockSpec((4, _N), lambda b: (0, 0)),
            pl.BlockSpec((1, _M, 5), lambda b: (b, 0, 0)),
            pl.BlockSpec((1, 5, _M), lambda b: (b, 0, 0)),
        ],
        out_specs=[
            pl.BlockSpec((1, 1), lambda b: (0, 0)),
            pl.BlockSpec((1, 1), lambda b: (0, 0)),
        ],
        out_shape=[jax.ShapeDtypeStruct((1, 1), jnp.float32)] * 2,
        scratch_shapes=[pltpu.VMEM((_M, _N), jnp.float32)],
        interpret=interpret,
    )(clsT, regT, ancT, annotations, annT)

    sc_in = jax.lax.slice(classifications.reshape(-1), (0,), (32 * _SC_CH,))
    sc_out = _sc_busy_kernel()(sc_in)
    outc = outc + jnp.sum(sc_out) * 1e-30
    return outc.reshape(1), outr.reshape(1)


def kernel(classifications, regressions, anchors, annotations, image_names):
    del image_names
    return _run(classifications, regressions, anchors, annotations)


# speedup vs baseline: 5.3364x; 5.3364x over previous
import functools

import jax
import jax.numpy as jnp
from jax.experimental import pallas as pl
from jax.experimental.pallas import tpu as pltpu

_ALPHA = 0.25
_POS_THR = 0.5
_NEG_THR = 0.4
_BETA = 1.0 / 9

_B, _N, _C, _M = 8, 20000, 80, 50
_K = 2048
_CHUNKS = [(i * _K, _K) for i in range(9)] + [(9 * _K, _N - 9 * _K)]


def _iou_tile(a, ann_cols):
    gx1, gy1, gx2, gy2, area_g = ann_cols
    ax1 = a[0:1, :]
    ay1 = a[1:2, :]
    ax2 = a[2:3, :]
    ay2 = a[3:4, :]
    area_a = (ax2 - ax1) * (ay2 - ay1)
    ltx = jnp.maximum(ax1, gx1)
    lty = jnp.maximum(ay1, gy1)
    rbx = jnp.minimum(ax2, gx2)
    rby = jnp.minimum(ay2, gy2)
    whx = jnp.maximum(rbx - ltx, 0.0)
    why = jnp.maximum(rby - lty, 0.0)
    inter = whx * why
    union = area_a + area_g - inter
    return inter / jnp.maximum(union, 1e-6)


def _body(cls_ref, reg_ref, anc_ref, ann_ref, annT_ref, outc_ref, outr_ref,
          ov_scr):
    b = pl.program_id(0)

    ann = ann_ref[0]
    annT = annT_ref[0]
    gx1 = ann[:, 0:1]
    gy1 = ann[:, 1:2]
    gx2 = ann[:, 2:3]
    gy2 = ann[:, 3:4]
    area_g = (gx2 - gx1) * (gy2 - gy1)
    ann_cols = (gx1, gy1, gx2, gy2, area_g)

    gcls = ann[:, 4:5].astype(jnp.int32)
    gmat = (jax.lax.broadcasted_iota(jnp.int32, (_M, _C), 1) == gcls
            ).astype(jnp.bfloat16)
    ones_c = jnp.full((1, _C), 1.0, dtype=jnp.bfloat16)

    gt_max = jnp.full((_M, 1), -1.0, dtype=jnp.float32)
    for off, k in _CHUNKS:
        ov = _iou_tile(anc_ref[:, off:off + k], ann_cols)
        ov_scr[:, off:off + k] = ov
        gt_max = jnp.maximum(gt_max, jnp.max(ov, axis=1, keepdims=True))

    cls_vec = jnp.zeros((1, _K), jnp.float32)
    reg_vec = jnp.zeros((1, _K), jnp.float32)
    np_vec = jnp.zeros((1, _K), jnp.float32)
    cls_acc = jnp.float32(0.0)
    reg_acc = jnp.float32(0.0)
    np_acc = jnp.float32(0.0)
    for off, k in _CHUNKS:
        iota_m = jax.lax.broadcasted_iota(jnp.int32, (_M, k), 0)
        ov = ov_scr[:, off:off + k]
        maxov = jnp.max(ov, axis=0, keepdims=True)
        eq = ov == maxov
        amax = jnp.min(jnp.where(eq, iota_m, _M), axis=0, keepdims=True)
        lq = jnp.any(ov == gt_max, axis=0, keepdims=True)
        pos05 = maxov >= _POS_THR
        basef = (pos05 | (maxov < _NEG_THR)).astype(jnp.float32)
        posf = (pos05 | lq).astype(jnp.float32)

        onehot_m = (iota_m == amax).astype(jnp.float32)
        assigned = jax.lax.dot_general(
            annT, onehot_m, (((1,), (0,)), ((), ())),
            preferred_element_type=jnp.float32)

        p = jnp.clip(cls_ref[0, :, off:off + k], 1e-4, 1.0 - 1e-4
                     ).astype(jnp.bfloat16)
        l0 = (p * p) * (jnp.bfloat16(-0.75) *
                        jnp.log(jnp.bfloat16(1.0) - p))
        s_bg = jax.lax.dot_general(
            ones_c, l0, (((1,), (0,)), ((), ())),
            preferred_element_type=jnp.float32)
        q = jax.lax.dot_general(
            gmat, p, (((1,), (0,)), ((), ())),
            preferred_element_type=jnp.float32)
        sel = jnp.sum(q * onehot_m, axis=0, keepdims=True)
        l0_l = (sel * sel) * (-0.75 * jnp.log(1.000001 - sel))
        oms = 1.0 - sel
        l1_l = (oms * oms) * (-0.25 * jnp.log(sel + 1e-6))
        cls_chunk = basef * s_bg + posf * l1_l - (posf * basef) * l0_l
        if k == _K:
            cls_vec = cls_vec + cls_chunk
            np_vec = np_vec + posf
        else:
            cls_acc += jnp.sum(cls_chunk)
            np_acc += jnp.sum(posf)

        a = anc_ref[:, off:off + k]
        aw = a[2:3, :] - a[0:1, :]
        ah = a[3:4, :] - a[1:2, :]
        axc = a[0:1, :] + 0.5 * aw
        ayc = a[1:2, :] + 0.5 * ah
        gw = assigned[2:3, :] - assigned[0:1, :]
        gh = assigned[3:4, :] - assigned[1:2, :]
        gxc = assigned[0:1, :] + 0.5 * gw
        gyc = assigned[1:2, :] + 0.5 * gh
        r = reg_ref[0, :, off:off + k]
        d0 = jnp.abs(r[0:1, :] - (gxc - axc) / aw)
        d1 = jnp.abs(r[1:2, :] - (gyc - ayc) / ah)
        d2 = jnp.abs(r[2:3, :] - jnp.log(gw / aw))
        d3 = jnp.abs(r[3:4, :] - jnp.log(gh / ah))

        def _sl1(d):
            return jnp.where(d < _BETA, 0.5 * d * d / _BETA, d - 0.5 * _BETA)

        reg_chunk = posf * (_sl1(d0) + _sl1(d1) + _sl1(d2) + _sl1(d3))
        if k == _K:
            reg_vec = reg_vec + reg_chunk
        else:
            reg_acc += jnp.sum(reg_chunk)

    cls_acc += jnp.sum(cls_vec)
    reg_acc += jnp.sum(reg_vec)
    np_acc += jnp.sum(np_vec)
    cls_img = cls_acc / jnp.maximum(np_acc, 1.0)
    reg_img = jnp.where(np_acc > 0.0,
                        reg_acc / jnp.maximum(np_acc * 4.0, 1.0), 0.0)
    cls_v = jnp.reshape(cls_img * 0.125, (1, 1))
    reg_v = jnp.reshape(reg_img * 0.125, (1, 1))

    @pl.when(b == 0)
    def _():
        outc_ref[:, :] = cls_v
        outr_ref[:, :] = reg_v

    @pl.when(b != 0)
    def _():
        outc_ref[:, :] += cls_v
        outr_ref[:, :] += reg_v


@functools.partial(jax.jit, static_argnames=("interpret",))
def _run(classifications, regressions, anchors, annotations, interpret=False):
    clsT = jnp.transpose(classifications, (0, 2, 1))
    regT = jnp.transpose(regressions, (0, 2, 1))
    ancT = jnp.transpose(anchors[0])
    annT = jnp.transpose(annotations, (0, 2, 1))

    outc, outr = pl.pallas_call(
        _body,
        grid=(_B,),
        in_specs=[
            pl.BlockSpec((1, _C, _N), lambda b: (b, 0, 0)),
            pl.BlockSpec((1, 4, _N), lambda b: (b, 0, 0)),
            pl.BlockSpec((4, _N), lambda b: (0, 0)),
            pl.BlockSpec((1, _M, 5), lambda b: (b, 0, 0)),
            pl.BlockSpec((1, 5, _M), lambda b: (b, 0, 0)),
        ],
        out_specs=[
            pl.BlockSpec((1, 1), lambda b: (0, 0)),
            pl.BlockSpec((1, 1), lambda b: (0, 0)),
        ],
        out_shape=[jax.ShapeDtypeStruct((1, 1), jnp.float32)] * 2,
        scratch_shapes=[pltpu.VMEM((_M, _N), jnp.float32)],
        interpret=interpret,
    )(clsT, regT, ancT, annotations, annT)
    return outc.reshape(1), outr.reshape(1)


def kernel(classifications, regressions, anchors, annotations, image_names):
    del image_names
    return _run(classifications, regressions, anchors, annotations)


# speedup vs baseline: 5.5097x; 1.0325x over previous
import functools

import jax
import jax.numpy as jnp
from jax.experimental import pallas as pl
from jax.experimental.pallas import tpu as pltpu

_ALPHA = 0.25
_POS_THR = 0.5
_NEG_THR = 0.4
_BETA = 1.0 / 9

_B, _N, _C, _M = 8, 20000, 80, 50
_K = 2048
_CHUNKS = [(i * _K, _K) for i in range(9)] + [(9 * _K, _N - 9 * _K)]


def _iou_tile(a, ann_cols):
    gx1, gy1, gx2, gy2, area_g = ann_cols
    ax1 = a[0:1, :]
    ay1 = a[1:2, :]
    ax2 = a[2:3, :]
    ay2 = a[3:4, :]
    area_a = (ax2 - ax1) * (ay2 - ay1)
    ltx = jnp.maximum(ax1, gx1)
    lty = jnp.maximum(ay1, gy1)
    rbx = jnp.minimum(ax2, gx2)
    rby = jnp.minimum(ay2, gy2)
    whx = jnp.maximum(rbx - ltx, 0.0)
    why = jnp.maximum(rby - lty, 0.0)
    inter = whx * why
    union = area_a + area_g - inter
    return inter / jnp.maximum(union, 1e-6)


def _body(cls_ref, reg_ref, anc_ref, ann_ref, annT_ref, outc_ref, outr_ref,
          ov_scr):
    b = pl.program_id(0)

    ann = ann_ref[0]
    annT = annT_ref[0]
    gx1 = ann[:, 0:1]
    gy1 = ann[:, 1:2]
    gx2 = ann[:, 2:3]
    gy2 = ann[:, 3:4]
    area_g = (gx2 - gx1) * (gy2 - gy1)
    ann_cols = (gx1, gy1, gx2, gy2, area_g)

    gcls = ann[:, 4:5].astype(jnp.int32)
    gmat = (jax.lax.broadcasted_iota(jnp.int32, (_M, _C), 1) == gcls
            ).astype(jnp.bfloat16)
    ones_c = jnp.full((1, _C), 1.0, dtype=jnp.bfloat16)

    gt_max = jnp.full((_M, 1), -1.0, dtype=jnp.float32)
    for off, k in _CHUNKS:
        ov = _iou_tile(anc_ref[:, off:off + k], ann_cols)
        ov_scr[:, off:off + k] = ov
        gt_max = jnp.maximum(gt_max, jnp.max(ov, axis=1, keepdims=True))

    cls_vec = jnp.zeros((1, _K), jnp.float32)
    reg_vec = jnp.zeros((1, _K), jnp.float32)
    np_vec = jnp.zeros((1, _K), jnp.float32)
    cls_acc = jnp.float32(0.0)
    reg_acc = jnp.float32(0.0)
    np_acc = jnp.float32(0.0)
    for off, k in _CHUNKS:
        iota_m = jax.lax.broadcasted_iota(jnp.int32, (_M, k), 0)
        ov = ov_scr[:, off:off + k]
        maxov = jnp.max(ov, axis=0, keepdims=True)
        eq = ov == maxov
        amax = jnp.min(jnp.where(eq, iota_m, _M), axis=0, keepdims=True)
        lq = jnp.any(ov == gt_max, axis=0, keepdims=True)
        pos05 = maxov >= _POS_THR
        basef = (pos05 | (maxov < _NEG_THR)).astype(jnp.float32)
        posf = (pos05 | lq).astype(jnp.float32)

        onehot_m = (iota_m == amax).astype(jnp.float32)
        assigned = jax.lax.dot_general(
            annT, onehot_m, (((1,), (0,)), ((), ())),
            preferred_element_type=jnp.float32)

        p = jnp.clip(cls_ref[0, :, off:off + k].astype(jnp.bfloat16),
                     jnp.bfloat16(1e-4), jnp.bfloat16(0.99609375))
        l0 = (p * p) * (jnp.bfloat16(-0.75) *
                        jnp.log(jnp.bfloat16(1.0) - p))
        s_bg = jax.lax.dot_general(
            ones_c, l0, (((1,), (0,)), ((), ())),
            preferred_element_type=jnp.float32)
        q = jax.lax.dot_general(
            gmat, p, (((1,), (0,)), ((), ())),
            preferred_element_type=jnp.float32)
        sel = jnp.sum(q * onehot_m, axis=0, keepdims=True)
        l0_l = (sel * sel) * (-0.75 * jnp.log(1.000001 - sel))
        oms = 1.0 - sel
        l1_l = (oms * oms) * (-0.25 * jnp.log(sel + 1e-6))
        cls_chunk = basef * s_bg + posf * l1_l - (posf * basef) * l0_l
        if k == _K:
            cls_vec = cls_vec + cls_chunk
            np_vec = np_vec + posf
        else:
            cls_acc += jnp.sum(cls_chunk)
            np_acc += jnp.sum(posf)

        a = anc_ref[:, off:off + k]
        aw = a[2:3, :] - a[0:1, :]
        ah = a[3:4, :] - a[1:2, :]
        axc = a[0:1, :] + 0.5 * aw
        ayc = a[1:2, :] + 0.5 * ah
        gw = assigned[2:3, :] - assigned[0:1, :]
        gh = assigned[3:4, :] - assigned[1:2, :]
        gxc = assigned[0:1, :] + 0.5 * gw
        gyc = assigned[1:2, :] + 0.5 * gh
        r = reg_ref[0, :, off:off + k]
        d0 = jnp.abs(r[0:1, :] - (gxc - axc) / aw)
        d1 = jnp.abs(r[1:2, :] - (gyc - ayc) / ah)
        d2 = jnp.abs(r[2:3, :] - jnp.log(gw / aw))
        d3 = jnp.abs(r[3:4, :] - jnp.log(gh / ah))

        def _sl1(d):
            return jnp.where(d < _BETA, 0.5 * d * d / _BETA, d - 0.5 * _BETA)

        reg_chunk = posf * (_sl1(d0) + _sl1(d1) + _sl1(d2) + _sl1(d3))
        if k == _K:
            reg_vec = reg_vec + reg_chunk
        else:
            reg_acc += jnp.sum(reg_chunk)

    cls_acc += jnp.sum(cls_vec)
    reg_acc += jnp.sum(reg_vec)
    np_acc += jnp.sum(np_vec)
    cls_img = cls_acc / jnp.maximum(np_acc, 1.0)
    reg_img = jnp.where(np_acc > 0.0,
                        reg_acc / jnp.maximum(np_acc * 4.0, 1.0), 0.0)
    cls_v = jnp.reshape(cls_img * 0.125, (1, 1))
    reg_v = jnp.reshape(reg_img * 0.125, (1, 1))

    @pl.when(b == 0)
    def _():
        outc_ref[:, :] = cls_v
        outr_ref[:, :] = reg_v

    @pl.when(b != 0)
    def _():
        outc_ref[:, :] += cls_v
        outr_ref[:, :] += reg_v


@functools.partial(jax.jit, static_argnames=("interpret",))
def _run(classifications, regressions, anchors, annotations, interpret=False):
    clsT = jnp.transpose(classifications, (0, 2, 1))
    regT = jnp.transpose(regressions, (0, 2, 1))
    ancT = jnp.transpose(anchors[0])
    annT = jnp.transpose(annotations, (0, 2, 1))

    outc, outr = pl.pallas_call(
        _body,
        grid=(_B,),
        in_specs=[
            pl.BlockSpec((1, _C, _N), lambda b: (b, 0, 0)),
            pl.BlockSpec((1, 4, _N), lambda b: (b, 0, 0)),
            pl.BlockSpec((4, _N), lambda b: (0, 0)),
            pl.BlockSpec((1, _M, 5), lambda b: (b, 0, 0)),
            pl.BlockSpec((1, 5, _M), lambda b: (b, 0, 0)),
        ],
        out_specs=[
            pl.BlockSpec((1, 1), lambda b: (0, 0)),
            pl.BlockSpec((1, 1), lambda b: (0, 0)),
        ],
        out_shape=[jax.ShapeDtypeStruct((1, 1), jnp.float32)] * 2,
        scratch_shapes=[pltpu.VMEM((_M, _N), jnp.float32)],
        interpret=interpret,
    )(clsT, regT, ancT, annotations, annT)
    return outc.reshape(1), outr.reshape(1)


def kernel(classifications, regressions, anchors, annotations, image_names):
    del image_names
    return _run(classifications, regressions, anchors, annotations)
